# Initial kernel scaffold; baseline (speedup 1.0000x reference)
import jax
import jax.numpy as jnp
from jax.experimental import pallas as pl


def kernel(observations, actions, W_trans, b_trans, g_init, enc_w1, enc_b1, enc_w2, enc_b2, dec_w1, dec_b1, dec_w2, dec_b2):
    raise NotImplementedError("write your pallas kernel here")


# speedup vs baseline: 38.9662x; 38.9662x over previous
import jax
import jax.numpy as jnp
from jax import lax
from jax.experimental import pallas as pl
from jax.experimental.pallas import tpu as pltpu

_ETA = 0.1


def _recur_body(oh_ref, wstack_ref, b_ref, ginit_ref, g_out_ref):
    s = g_out_ref.shape[0]
    bb = g_out_ref.shape[1]
    h = g_out_ref.shape[2]
    a = oh_ref.shape[2]
    g0 = jnp.broadcast_to(ginit_ref[...], (bb, h))
    g_out_ref[0:1] = g0[None]

    def step(t, g):
        oh = oh_ref[t - 1]
        gext = jnp.concatenate(
            [oh[:, i:i + 1] * g for i in range(a)], axis=1)
        z = jnp.dot(gext, wstack_ref[...],
                    preferred_element_type=jnp.float32)
        g2 = jnp.tanh(z + b_ref[...])
        g_out_ref[pl.ds(t, 1)] = g2[None]
        return g2

    lax.fori_loop(1, s, step, g0)


def _fused_body(obs_ref, g_ref, ew1_ref, eb1_ref, ew2_ref, eb2_ref,
                dw1_ref, db1_ref, dw2_ref, db2_ref, out_ref):
    nrows = obs_ref.shape[0]
    s = g_ref.shape[0]
    nb = g_ref.shape[1]
    chunk = 256

    p_parts = []
    for r in range(0, nrows, chunk):
        x = obs_ref[r:r + chunk]
        hh = jnp.maximum(
            jnp.dot(x, ew1_ref[...], preferred_element_type=jnp.float32)
            + eb1_ref[...], 0.0)
        p_parts.append(
            jnp.dot(hh, ew2_ref[...], preferred_element_type=jnp.float32)
            + eb2_ref[...])

    it = lax.broadcasted_iota(jnp.int32, (s, s), 0)
    ik = lax.broadcasted_iota(jnp.int32, (s, s), 1)
    wmask = jnp.where(ik < it, _ETA, 0.0)
    sel0 = (it + ik) == 0
    ps_parts = []
    per = chunk // s
    for j in range(nb):
        gj = g_ref[:, j, :]
        pj = p_parts[j // per][(j % per) * s:(j % per) * s + s]
        sc = lax.dot_general(gj, gj, (((1,), (1,)), ((), ())),
                             preferred_element_type=jnp.float32)
        scm = jnp.where(sel0, 1.0, sc * wmask)
        ps_parts.append(
            jnp.dot(scm, pj, preferred_element_type=jnp.float32))

    for c in range(0, nb, per):
        pseq = jnp.concatenate(ps_parts[c:c + per], axis=0)
        h2 = jnp.maximum(
            jnp.dot(pseq, dw1_ref[...], preferred_element_type=jnp.float32)
            + db1_ref[...], 0.0)
        out_ref[c * s:(c + per) * s] = (
            jnp.dot(h2, dw2_ref[...], preferred_element_type=jnp.float32)
            + db2_ref[...])


def kernel(observations, actions, W_trans, b_trans, g_init,
           enc_w1, enc_b1, enc_w2, enc_b2,
           dec_w1, dec_b1, dec_w2, dec_b2):
    b, s, d = observations.shape
    h = g_init.shape[0]
    a = W_trans.shape[0]
    h2 = enc_w1.shape[1]

    obs2 = observations.reshape(b * s, d)
    oh_sb = jnp.transpose(
        jax.nn.one_hot(actions, a, dtype=jnp.float32), (1, 0, 2))
    wstack = jnp.transpose(W_trans, (0, 2, 1)).reshape(a * h, h)

    nb_rec = 2
    bb = b // nb_rec
    g_seq = pl.pallas_call(
        _recur_body,
        grid=(nb_rec,),
        in_specs=[
            pl.BlockSpec((s, bb, a), lambda i: (0, i, 0)),
            pl.BlockSpec((a * h, h), lambda i: (0, 0)),
            pl.BlockSpec((1, h), lambda i: (0, 0)),
            pl.BlockSpec((1, h), lambda i: (0, 0)),
        ],
        out_specs=pl.BlockSpec((s, bb, h), lambda i: (0, i, 0)),
        out_shape=jax.ShapeDtypeStruct((s, b, h), jnp.float32),
        compiler_params=pltpu.CompilerParams(
            dimension_semantics=("parallel",)),
    )(oh_sb, wstack, b_trans.reshape(1, h), g_init.reshape(1, h))

    blk_b = 8
    rows = blk_b * s
    out2 = pl.pallas_call(
        _fused_body,
        grid=(b // blk_b,),
        in_specs=[
            pl.BlockSpec((rows, d), lambda i: (i, 0)),
            pl.BlockSpec((s, blk_b, h), lambda i: (0, i, 0)),
            pl.BlockSpec((d, h2), lambda i: (0, 0)),
            pl.BlockSpec((1, h2), lambda i: (0, 0)),
            pl.BlockSpec((h2, h), lambda i: (0, 0)),
            pl.BlockSpec((1, h), lambda i: (0, 0)),
            pl.BlockSpec((h, h2), lambda i: (0, 0)),
            pl.BlockSpec((1, h2), lambda i: (0, 0)),
            pl.BlockSpec((h2, d), lambda i: (0, 0)),
            pl.BlockSpec((1, d), lambda i: (0, 0)),
        ],
        out_specs=pl.BlockSpec((rows, d), lambda i: (i, 0)),
        out_shape=jax.ShapeDtypeStruct((b * s, d), jnp.float32),
        compiler_params=pltpu.CompilerParams(
            dimension_semantics=("parallel",)),
    )(obs2, g_seq, enc_w1, enc_b1.reshape(1, h2), enc_w2,
      enc_b2.reshape(1, h), dec_w1, dec_b1.reshape(1, h2), dec_w2,
      dec_b2.reshape(1, d))
    return out2.reshape(b, s, d)


# speedup vs baseline: 40.2539x; 1.0330x over previous
import jax
import jax.numpy as jnp
from jax import lax
from jax.experimental import pallas as pl
from jax.experimental.pallas import tpu as pltpu

_ETA = 0.1


def _recur_body(oh_ref, wstack_ref, b_ref, ginit_ref, g_out_ref):
    s = g_out_ref.shape[0]
    bb = g_out_ref.shape[1]
    h = g_out_ref.shape[2]
    a = oh_ref.shape[2]
    g0 = jnp.broadcast_to(ginit_ref[...], (bb, h))
    g_out_ref[0:1] = g0[None]

    def step(t, g):
        oh = oh_ref[t - 1]
        gb = g.astype(jnp.bfloat16)
        gext = jnp.concatenate(
            [oh[:, i:i + 1] * gb for i in range(a)], axis=1)
        z = jnp.dot(gext, wstack_ref[...],
                    preferred_element_type=jnp.float32)
        g2 = jnp.tanh(z + b_ref[...])
        g_out_ref[pl.ds(t, 1)] = g2[None]
        return g2

    lax.fori_loop(1, s, step, g0)


def _fused_body(obs_ref, g_ref, ew1_ref, eb1_ref, ew2_ref, eb2_ref,
                dw1_ref, db1_ref, dw2_ref, db2_ref, out_ref):
    nrows = obs_ref.shape[0]
    s = g_ref.shape[0]
    nb = g_ref.shape[1]
    chunk = 256

    p_parts = []
    for r in range(0, nrows, chunk):
        x = obs_ref[r:r + chunk]
        hh = jnp.maximum(
            jnp.dot(x, ew1_ref[...], preferred_element_type=jnp.float32)
            + eb1_ref[...], 0.0)
        p_parts.append(
            jnp.dot(hh, ew2_ref[...], preferred_element_type=jnp.float32)
            + eb2_ref[...])

    it = lax.broadcasted_iota(jnp.int32, (s, s), 0)
    ik = lax.broadcasted_iota(jnp.int32, (s, s), 1)
    wmask = jnp.where(ik < it, _ETA, 0.0)
    sel0 = (it + ik) == 0
    ps_parts = []
    per = chunk // s
    for j in range(nb):
        gj = g_ref[:, j, :]
        pj = p_parts[j // per][(j % per) * s:(j % per) * s + s]
        sc = lax.dot_general(gj, gj, (((1,), (1,)), ((), ())),
                             preferred_element_type=jnp.float32)
        scm = jnp.where(sel0, 1.0, sc * wmask)
        ps_parts.append(
            jnp.dot(scm, pj, preferred_element_type=jnp.float32))

    for c in range(0, nb, per):
        pseq = jnp.concatenate(ps_parts[c:c + per], axis=0)
        h2 = jnp.maximum(
            jnp.dot(pseq, dw1_ref[...], preferred_element_type=jnp.float32)
            + db1_ref[...], 0.0)
        out_ref[c * s:(c + per) * s] = (
            jnp.dot(h2, dw2_ref[...], preferred_element_type=jnp.float32)
            + db2_ref[...])


def kernel(observations, actions, W_trans, b_trans, g_init,
           enc_w1, enc_b1, enc_w2, enc_b2,
           dec_w1, dec_b1, dec_w2, dec_b2):
    b, s, d = observations.shape
    h = g_init.shape[0]
    a = W_trans.shape[0]
    h2 = enc_w1.shape[1]

    obs2 = observations.reshape(b * s, d)
    oh_sb = jnp.transpose(
        jax.nn.one_hot(actions, a, dtype=jnp.bfloat16), (1, 0, 2))
    wstack = jnp.transpose(W_trans, (0, 2, 1)).reshape(
        a * h, h).astype(jnp.bfloat16)

    nb_rec = 2
    bb = b // nb_rec
    g_seq = pl.pallas_call(
        _recur_body,
        grid=(nb_rec,),
        in_specs=[
            pl.BlockSpec((s, bb, a), lambda i: (0, i, 0)),
            pl.BlockSpec((a * h, h), lambda i: (0, 0)),

            pl.BlockSpec((1, h), lambda i: (0, 0)),
            pl.BlockSpec((1, h), lambda i: (0, 0)),
        ],
        out_specs=pl.BlockSpec((s, bb, h), lambda i: (0, i, 0)),
        out_shape=jax.ShapeDtypeStruct((s, b, h), jnp.float32),
        compiler_params=pltpu.CompilerParams(
            dimension_semantics=("parallel",)),
    )(oh_sb, wstack, b_trans.reshape(1, h), g_init.reshape(1, h))

    blk_b = 8
    rows = blk_b * s
    out2 = pl.pallas_call(
        _fused_body,
        grid=(b // blk_b,),
        in_specs=[
            pl.BlockSpec((rows, d), lambda i: (i, 0)),
            pl.BlockSpec((s, blk_b, h), lambda i: (0, i, 0)),
            pl.BlockSpec((d, h2), lambda i: (0, 0)),
            pl.BlockSpec((1, h2), lambda i: (0, 0)),
            pl.BlockSpec((h2, h), lambda i: (0, 0)),
            pl.BlockSpec((1, h), lambda i: (0, 0)),
            pl.BlockSpec((h, h2), lambda i: (0, 0)),
            pl.BlockSpec((1, h2), lambda i: (0, 0)),
            pl.BlockSpec((h2, d), lambda i: (0, 0)),
            pl.BlockSpec((1, d), lambda i: (0, 0)),
        ],
        out_specs=pl.BlockSpec((rows, d), lambda i: (i, 0)),
        out_shape=jax.ShapeDtypeStruct((b * s, d), jnp.float32),
        compiler_params=pltpu.CompilerParams(
            dimension_semantics=("parallel",)),
    )(obs2, g_seq, enc_w1, enc_b1.reshape(1, h2), enc_w2,
      enc_b2.reshape(1, h), dec_w1, dec_b1.reshape(1, h2), dec_w2,
      dec_b2.reshape(1, d))
    return out2.reshape(b, s, d)


# speedup vs baseline: 41.6597x; 1.0349x over previous
import jax
import jax.numpy as jnp
from jax import lax
from jax.experimental import pallas as pl
from jax.experimental.pallas import tpu as pltpu

_ETA = 0.1


def _recur_body(oh_ref, wstack_ref, b_ref, ginit_ref, g_out_ref):
    s = g_out_ref.shape[0]
    bb = g_out_ref.shape[1]
    h = g_out_ref.shape[2]
    a = oh_ref.shape[2]
    g0 = jnp.broadcast_to(ginit_ref[...], (bb, h))
    g_out_ref[0:1] = g0[None]

    def one(t, g):
        oh = oh_ref[t - 1]
        gb = g.astype(jnp.bfloat16)
        gext = jnp.concatenate(
            [oh[:, i:i + 1] * gb for i in range(a)], axis=1)
        z = jnp.dot(gext, wstack_ref[...],
                    preferred_element_type=jnp.float32)
        g2 = jnp.tanh(z + b_ref[...])
        g_out_ref[pl.ds(t, 1)] = g2[None]
        return g2

    def step(i, g):
        t = 1 + i * 2
        return one(t + 1, one(t, g))

    g_last = lax.fori_loop(0, (s - 2) // 2, step, g0)
    one(s - 1, g_last)


def _fused_body(obs_ref, g_ref, ew1_ref, eb1_ref, ew2_ref, eb2_ref,
                dw1_ref, db1_ref, dw2_ref, db2_ref, out_ref):
    nrows = obs_ref.shape[0]
    s = g_ref.shape[0]
    nb = g_ref.shape[1]
    chunk = 256

    p_parts = []
    for r in range(0, nrows, chunk):
        x = obs_ref[r:r + chunk]
        hh = jnp.maximum(
            jnp.dot(x, ew1_ref[...], preferred_element_type=jnp.float32)
            + eb1_ref[...], 0.0)
        p_parts.append(
            jnp.dot(hh, ew2_ref[...], preferred_element_type=jnp.float32)
            + eb2_ref[...])

    it = lax.broadcasted_iota(jnp.int32, (s, s), 0)
    ik = lax.broadcasted_iota(jnp.int32, (s, s), 1)
    wmask = jnp.where(ik < it, _ETA, 0.0)
    sel0 = (it + ik) == 0
    ps_parts = []
    per = chunk // s
    for j in range(nb):
        gj = g_ref[:, j, :]
        pj = p_parts[j // per][(j % per) * s:(j % per) * s + s]
        sc = lax.dot_general(gj, gj, (((1,), (1,)), ((), ())),
                             preferred_element_type=jnp.float32)
        scm = jnp.where(sel0, 1.0, sc * wmask)
        ps_parts.append(
            jnp.dot(scm, pj, preferred_element_type=jnp.float32))

    for c in range(0, nb, per):
        pseq = jnp.concatenate(ps_parts[c:c + per], axis=0)
        h2 = jnp.maximum(
            jnp.dot(pseq, dw1_ref[...], preferred_element_type=jnp.float32)
            + db1_ref[...], 0.0)
        out_ref[c * s:(c + per) * s] = (
            jnp.dot(h2, dw2_ref[...], preferred_element_type=jnp.float32)
            + db2_ref[...])


def kernel(observations, actions, W_trans, b_trans, g_init,
           enc_w1, enc_b1, enc_w2, enc_b2,
           dec_w1, dec_b1, dec_w2, dec_b2):
    b, s, d = observations.shape
    h = g_init.shape[0]
    a = W_trans.shape[0]
    h2 = enc_w1.shape[1]

    obs2 = observations.reshape(b * s, d)
    oh_sb = jnp.transpose(
        jax.nn.one_hot(actions, a, dtype=jnp.bfloat16), (1, 0, 2))
    wstack = jnp.transpose(W_trans, (0, 2, 1)).reshape(
        a * h, h).astype(jnp.bfloat16)

    nb_rec = 2
    bb = b // nb_rec
    g_seq = pl.pallas_call(
        _recur_body,
        grid=(nb_rec,),
        in_specs=[
            pl.BlockSpec((s, bb, a), lambda i: (0, i, 0)),
            pl.BlockSpec((a * h, h), lambda i: (0, 0)),

            pl.BlockSpec((1, h), lambda i: (0, 0)),
            pl.BlockSpec((1, h), lambda i: (0, 0)),
        ],
        out_specs=pl.BlockSpec((s, bb, h), lambda i: (0, i, 0)),
        out_shape=jax.ShapeDtypeStruct((s, b, h), jnp.float32),
        compiler_params=pltpu.CompilerParams(
            dimension_semantics=("parallel",)),
    )(oh_sb, wstack, b_trans.reshape(1, h), g_init.reshape(1, h))

    blk_b = 8
    rows = blk_b * s
    out2 = pl.pallas_call(
        _fused_body,
        grid=(b // blk_b,),
        in_specs=[
            pl.BlockSpec((rows, d), lambda i: (i, 0)),
            pl.BlockSpec((s, blk_b, h), lambda i: (0, i, 0)),
            pl.BlockSpec((d, h2), lambda i: (0, 0)),
            pl.BlockSpec((1, h2), lambda i: (0, 0)),
            pl.BlockSpec((h2, h), lambda i: (0, 0)),
            pl.BlockSpec((1, h), lambda i: (0, 0)),
            pl.BlockSpec((h, h2), lambda i: (0, 0)),
            pl.BlockSpec((1, h2), lambda i: (0, 0)),
            pl.BlockSpec((h2, d), lambda i: (0, 0)),
            pl.BlockSpec((1, d), lambda i: (0, 0)),
        ],
        out_specs=pl.BlockSpec((rows, d), lambda i: (i, 0)),
        out_shape=jax.ShapeDtypeStruct((b * s, d), jnp.float32),
        compiler_params=pltpu.CompilerParams(
            dimension_semantics=("parallel",)),
    )(obs2, g_seq, enc_w1, enc_b1.reshape(1, h2), enc_w2,
      enc_b2.reshape(1, h), dec_w1, dec_b1.reshape(1, h2), dec_w2,
      dec_b2.reshape(1, d))
    return out2.reshape(b, s, d)
